# baseline (speedup 1.0000x reference)
import functools

import jax
import jax.numpy as jnp
from jax import lax
from jax.experimental import pallas as pl
from jax.experimental.pallas import tpu as pltpu
from jax.experimental.pallas import tpu_sc as plsc

N_FEATURES = 26
ROWS_PER_FEATURE = 2
W_DIM = 16
B, T = 1024, 20
BT = B * T

NC, NS, L = 2, 16, 16
NW = NC * NS
ROWS_PER_W = BT // NW
IDX_MINOR = 128
IDX_MAJOR = ROWS_PER_W // IDX_MINOR
D = ROWS_PER_FEATURE * W_DIM
NBUF = 2

VOCAB = 2600000
VB = 4096


NBLK = (VOCAB + VB - 1) // VB
VOCAB_PAD = NBLK * VB


def _tc_linearize(table_t):
    def body(x_ref, o_ref):
        y = jnp.transpose(x_ref[...])
        for c in range(VB // 1024):
            o_ref[:, 32 * c:32 * (c + 1)] = y[1024 * c:1024 * (c + 1), :]

    return pl.pallas_call(
        body,
        grid=(NBLK,),
        in_specs=[pl.BlockSpec((32, VB), lambda i: (0, i))],
        out_specs=pl.BlockSpec((VB // 4, 128), lambda i: (i, 0)),
        out_shape=jax.ShapeDtypeStruct((VOCAB_PAD * D // 128, 128), jnp.float32),
    )(table_t)


def _sc_gather(feats, offsets_pad, table):
    mesh = plsc.VectorSubcoreMesh(
        core_axis_name="c", subcore_axis_name="s", num_cores=NC, num_subcores=NS
    )

    @functools.partial(
        pl.kernel,
        out_type=jax.ShapeDtypeStruct((BT, N_FEATURES * D), jnp.float32),
        mesh=mesh,
        scratch_types=[
            pltpu.VMEM((IDX_MAJOR, N_FEATURES, IDX_MINOR), jnp.int32),
            pltpu.VMEM((NBUF, ROWS_PER_W, D), jnp.float32),
            pltpu.VMEM((32,), jnp.int32),
            pltpu.SemaphoreType.DMA,
            pltpu.SemaphoreType.DMA,
        ],
        compiler_params=pltpu.CompilerParams(
            use_tc_tiling_on_sc=False, needs_layout_passes=False
        ),
    )
    def k(feats_hbm, off_hbm, table_hbm, out_hbm, idx_v, rows_v, off_v, sem0, sem1):
        wid = lax.axis_index("s") * NC + lax.axis_index("c")
        gsems = [sem0, sem1]
        pltpu.sync_copy(off_hbm, off_v)
        pltpu.sync_copy(feats_hbm.at[pl.ds(wid * IDX_MAJOR, IDX_MAJOR)], idx_v)

        def add_feature(f, _):
            off_vec = plsc.load_gather(off_v, [jnp.full((L,), f, jnp.int32)])
            for j in range(IDX_MAJOR):
                for i in range(IDX_MINOR // L):
                    sl = (j, f, pl.ds(i * L, L))
                    v = idx_v[sl] + off_vec
                    t = v & 4095
                    idx_v[sl] = (v - t) + ((t & 1023) << 2) + (t >> 10)
            return 0

        lax.fori_loop(0, N_FEATURES, add_feature, 0)

        def fire(f, b):
            for j in range(IDX_MAJOR):
                pltpu.async_copy(
                    table_hbm.at[idx_v.at[j, f]],
                    rows_v.at[b, pl.ds(j * IDX_MINOR, IDX_MINOR)],
                    gsems[b],
                )

        def drain(b):
            pltpu.make_async_copy(
                table_hbm.at[pl.ds(0, ROWS_PER_W)], rows_v.at[b], gsems[b]
            ).wait()

        fire(0, 0)
        fire(1, 1)

        def step(i, _):
            for b in range(NBUF):
                f = i * NBUF + b
                drain(b)
                pltpu.sync_copy(
                    rows_v.at[b],
                    out_hbm.at[
                        pl.ds(wid * ROWS_PER_W, ROWS_PER_W), pl.ds(f * D, D)
                    ],
                )

                @pl.when(f + NBUF < N_FEATURES)
                def _():
                    fire(f + NBUF, b)

            return 0

        lax.fori_loop(0, N_FEATURES // NBUF, step, 0)

    return k(feats, offsets_pad, table)


def kernel(f0, f1, f2, f3, f4, f5, f6, f7, f8, f9, f10, f11, f12, f13, f14,
           f15, f16, f17, f18, f19, f20, f21, f22, f23, f24, f25, offsets,
           embed_weight):
    features = [f0, f1, f2, f3, f4, f5, f6, f7, f8, f9, f10, f11, f12, f13,
                f14, f15, f16, f17, f18, f19, f20, f21, f22, f23, f24, f25]
    feats = (
        jnp.stack(features, axis=2)
        .transpose(2, 1, 0)
        .reshape(N_FEATURES, BT // IDX_MINOR, IDX_MINOR)
        .transpose(1, 0, 2)
    )
    offsets_pad = jnp.pad(offsets.astype(jnp.int32), (0, 32 - N_FEATURES))
    table_lin = _tc_linearize(embed_weight.T).reshape(VOCAB_PAD, D)
    out = _sc_gather(feats, offsets_pad, table_lin)
    return (
        out.reshape(T, B, N_FEATURES * ROWS_PER_FEATURE, W_DIM)
        .transpose(1, 0, 2, 3)
    )


# speedup vs baseline: 1.0984x; 1.0984x over previous
import functools

import jax
import jax.numpy as jnp
from jax import lax
from jax.experimental import pallas as pl
from jax.experimental.pallas import tpu as pltpu
from jax.experimental.pallas import tpu_sc as plsc

N_FEATURES = 26
ROWS_PER_FEATURE = 2
W_DIM = 16
B, T = 1024, 20
BT = B * T

NC, NS, L = 2, 16, 16
NW = NC * NS
ROWS_PER_W = BT // NW
IDX_MINOR = 128
IDX_MAJOR = ROWS_PER_W // IDX_MINOR
D = ROWS_PER_FEATURE * W_DIM
NBUF = 2

VOCAB = 2600000
VB = 8192
QROWS = VB // 4
QSHIFT = 11


NBLK = (VOCAB + VB - 1) // VB
VOCAB_PAD = NBLK * VB


def _tc_linearize(table_t):
    def body(x_ref, o_ref):
        y = jnp.transpose(x_ref[...])
        for c in range(4):
            o_ref[:, 32 * c:32 * (c + 1)] = y[QROWS * c:QROWS * (c + 1), :]

    return pl.pallas_call(
        body,
        grid=(NBLK,),
        in_specs=[pl.BlockSpec((32, VB), lambda i: (0, i))],
        out_specs=pl.BlockSpec((VB // 4, 128), lambda i: (i, 0)),
        out_shape=jax.ShapeDtypeStruct((VOCAB_PAD * D // 128, 128), jnp.float32),
    )(table_t)


def _sc_gather(feats, offsets_pad, table):
    mesh = plsc.VectorSubcoreMesh(
        core_axis_name="c", subcore_axis_name="s", num_cores=NC, num_subcores=NS
    )

    @functools.partial(
        pl.kernel,
        out_type=jax.ShapeDtypeStruct((BT, N_FEATURES * D), jnp.float32),
        mesh=mesh,
        scratch_types=[
            pltpu.VMEM((IDX_MAJOR, N_FEATURES, IDX_MINOR), jnp.int32),
            pltpu.VMEM((NBUF, ROWS_PER_W, D), jnp.float32),
            pltpu.VMEM((32,), jnp.int32),
            pltpu.SemaphoreType.DMA,
            pltpu.SemaphoreType.DMA,
        ],
        compiler_params=pltpu.CompilerParams(
            use_tc_tiling_on_sc=False, needs_layout_passes=False
        ),
    )
    def k(feats_hbm, off_hbm, table_hbm, out_hbm, idx_v, rows_v, off_v, sem0, sem1):
        wid = lax.axis_index("s") * NC + lax.axis_index("c")
        gsems = [sem0, sem1]
        pltpu.sync_copy(off_hbm, off_v)
        pltpu.sync_copy(feats_hbm.at[pl.ds(wid * IDX_MAJOR, IDX_MAJOR)], idx_v)

        def add_feature(f, _):
            off_vec = plsc.load_gather(off_v, [jnp.full((L,), f, jnp.int32)])
            for j in range(IDX_MAJOR):
                for i in range(IDX_MINOR // L):
                    sl = (j, f, pl.ds(i * L, L))
                    v = idx_v[sl] + off_vec
                    t = v & (VB - 1)
                    idx_v[sl] = (v - t) + ((t & (QROWS - 1)) << 2) + (t >> QSHIFT)
            return 0

        lax.fori_loop(0, N_FEATURES, add_feature, 0)

        def fire(f, b):
            for j in range(IDX_MAJOR):
                pltpu.async_copy(
                    table_hbm.at[idx_v.at[j, f]],
                    rows_v.at[b, pl.ds(j * IDX_MINOR, IDX_MINOR)],
                    gsems[b],
                )

        def drain(b):
            pltpu.make_async_copy(
                table_hbm.at[pl.ds(0, ROWS_PER_W)], rows_v.at[b], gsems[b]
            ).wait()

        fire(0, 0)
        fire(1, 1)

        def step(i, _):
            for b in range(NBUF):
                f = i * NBUF + b
                drain(b)
                pltpu.sync_copy(
                    rows_v.at[b],
                    out_hbm.at[
                        pl.ds(wid * ROWS_PER_W, ROWS_PER_W), pl.ds(f * D, D)
                    ],
                )

                @pl.when(f + NBUF < N_FEATURES)
                def _():
                    fire(f + NBUF, b)

            return 0

        lax.fori_loop(0, N_FEATURES // NBUF, step, 0)

    return k(feats, offsets_pad, table)


def kernel(f0, f1, f2, f3, f4, f5, f6, f7, f8, f9, f10, f11, f12, f13, f14,
           f15, f16, f17, f18, f19, f20, f21, f22, f23, f24, f25, offsets,
           embed_weight):
    features = [f0, f1, f2, f3, f4, f5, f6, f7, f8, f9, f10, f11, f12, f13,
                f14, f15, f16, f17, f18, f19, f20, f21, f22, f23, f24, f25]
    feats = (
        jnp.stack(features, axis=2)
        .transpose(2, 1, 0)
        .reshape(N_FEATURES, BT // IDX_MINOR, IDX_MINOR)
        .transpose(1, 0, 2)
    )
    offsets_pad = jnp.pad(offsets.astype(jnp.int32), (0, 32 - N_FEATURES))
    table_lin = _tc_linearize(embed_weight.T).reshape(VOCAB_PAD, D)
    out = _sc_gather(feats, offsets_pad, table_lin)
    return (
        out.reshape(T, B, N_FEATURES * ROWS_PER_FEATURE, W_DIM)
        .transpose(1, 0, 2, 3)
    )


# speedup vs baseline: 1.1143x; 1.0145x over previous
import functools

import jax
import jax.numpy as jnp
from jax import lax
from jax.experimental import pallas as pl
from jax.experimental.pallas import tpu as pltpu
from jax.experimental.pallas import tpu_sc as plsc

N_FEATURES = 26
ROWS_PER_FEATURE = 2
W_DIM = 16
B, T = 1024, 20
BT = B * T

NC, NS, L = 2, 16, 16
NW = NC * NS
ROWS_PER_W = BT // NW
IDX_MINOR = 128
IDX_MAJOR = ROWS_PER_W // IDX_MINOR
D = ROWS_PER_FEATURE * W_DIM
NBUF = 2

VOCAB = 2600000
VB = 16384
QROWS = VB // 4
QSHIFT = 12


NBLK = (VOCAB + VB - 1) // VB
VOCAB_PAD = NBLK * VB


def _tc_linearize(table_t):
    def body(x_ref, o_ref):
        y = jnp.transpose(x_ref[...])
        for c in range(4):
            o_ref[:, 32 * c:32 * (c + 1)] = y[QROWS * c:QROWS * (c + 1), :]

    return pl.pallas_call(
        body,
        grid=(NBLK,),
        in_specs=[pl.BlockSpec((32, VB), lambda i: (0, i))],
        out_specs=pl.BlockSpec((VB // 4, 128), lambda i: (i, 0)),
        out_shape=jax.ShapeDtypeStruct((VOCAB_PAD * D // 128, 128), jnp.float32),
    )(table_t)


def _sc_gather(feats, offsets_pad, table):
    mesh = plsc.VectorSubcoreMesh(
        core_axis_name="c", subcore_axis_name="s", num_cores=NC, num_subcores=NS
    )

    @functools.partial(
        pl.kernel,
        out_type=jax.ShapeDtypeStruct((BT, N_FEATURES * D), jnp.float32),
        mesh=mesh,
        scratch_types=[
            pltpu.VMEM((IDX_MAJOR, N_FEATURES, IDX_MINOR), jnp.int32),
            pltpu.VMEM((NBUF, ROWS_PER_W, D), jnp.float32),
            pltpu.VMEM((32,), jnp.int32),
            pltpu.SemaphoreType.DMA,
            pltpu.SemaphoreType.DMA,
        ],
        compiler_params=pltpu.CompilerParams(
            use_tc_tiling_on_sc=False, needs_layout_passes=False
        ),
    )
    def k(feats_hbm, off_hbm, table_hbm, out_hbm, idx_v, rows_v, off_v, sem0, sem1):
        wid = lax.axis_index("s") * NC + lax.axis_index("c")
        gsems = [sem0, sem1]
        pltpu.sync_copy(off_hbm, off_v)
        pltpu.sync_copy(feats_hbm.at[pl.ds(wid * IDX_MAJOR, IDX_MAJOR)], idx_v)

        def add_feature(f, _):
            off_vec = plsc.load_gather(off_v, [jnp.full((L,), f, jnp.int32)])
            for j in range(IDX_MAJOR):
                for i in range(IDX_MINOR // L):
                    sl = (j, f, pl.ds(i * L, L))
                    v = idx_v[sl] + off_vec
                    t = v & (VB - 1)
                    idx_v[sl] = (v - t) + ((t & (QROWS - 1)) << 2) + (t >> QSHIFT)
            return 0

        lax.fori_loop(0, N_FEATURES, add_feature, 0)

        def fire(f, b):
            for j in range(IDX_MAJOR):
                pltpu.async_copy(
                    table_hbm.at[idx_v.at[j, f]],
                    rows_v.at[b, pl.ds(j * IDX_MINOR, IDX_MINOR)],
                    gsems[b],
                )

        def drain(b):
            pltpu.make_async_copy(
                table_hbm.at[pl.ds(0, ROWS_PER_W)], rows_v.at[b], gsems[b]
            ).wait()

        fire(0, 0)
        fire(1, 1)

        def step(i, _):
            for b in range(NBUF):
                f = i * NBUF + b
                drain(b)
                pltpu.sync_copy(
                    rows_v.at[b],
                    out_hbm.at[
                        pl.ds(wid * ROWS_PER_W, ROWS_PER_W), pl.ds(f * D, D)
                    ],
                )

                @pl.when(f + NBUF < N_FEATURES)
                def _():
                    fire(f + NBUF, b)

            return 0

        lax.fori_loop(0, N_FEATURES // NBUF, step, 0)

    return k(feats, offsets_pad, table)


def kernel(f0, f1, f2, f3, f4, f5, f6, f7, f8, f9, f10, f11, f12, f13, f14,
           f15, f16, f17, f18, f19, f20, f21, f22, f23, f24, f25, offsets,
           embed_weight):
    features = [f0, f1, f2, f3, f4, f5, f6, f7, f8, f9, f10, f11, f12, f13,
                f14, f15, f16, f17, f18, f19, f20, f21, f22, f23, f24, f25]
    feats = (
        jnp.stack(features, axis=2)
        .transpose(2, 1, 0)
        .reshape(N_FEATURES, BT // IDX_MINOR, IDX_MINOR)
        .transpose(1, 0, 2)
    )
    offsets_pad = jnp.pad(offsets.astype(jnp.int32), (0, 32 - N_FEATURES))
    table_lin = _tc_linearize(embed_weight.T).reshape(VOCAB_PAD, D)
    out = _sc_gather(feats, offsets_pad, table_lin)
    return (
        out.reshape(T, B, N_FEATURES * ROWS_PER_FEATURE, W_DIM)
        .transpose(1, 0, 2, 3)
    )
